# baseline (speedup 1.0000x reference)
import functools

import jax
import jax.numpy as jnp
from jax import lax
from jax.experimental import pallas as pl
from jax.experimental.pallas import tpu as pltpu
from jax.experimental.pallas import tpu_sc as plsc

_B = 4096
_S = 200
_V2 = 1000
_HALF = 32
_EMBED = 64
_NC = 2
_NS = 16
_NW = _NC * _NS
_BT = _B // _NW
_L = 16
_NBUF = 3
_OPITCH = 129


def _compute(r1, t2_v, idx2_v, pe_v, ob, s, iota):
    pev = []
    etv = []
    eiv = []
    for g in range(_EMBED // _L):
        pev.append(pe_v[s, pl.ds(g * _L, _L)])
        ev = iota + g * _L
        etv.append(ev >> 3)
        eiv.append(ev & 7)

    @plsc.parallel_loop(0, _BT // _L)
    def _chunk(c):
        tokv = idx2_v[s, pl.ds(c * _L, _L)]
        for j in range(_L):
            b = c * _L + j
            bv = jnp.full((_L,), b, jnp.int32)
            tok = tokv[j]
            for g in range(2):
                v = r1[b, pl.ds(g * _L, _L)] + pev[g]
                plsc.store_scatter(ob, [etv[g], eiv[g], bv], v)
            for g in range(2):
                v = t2_v[tok, pl.ds(g * _L, _L)] + pev[2 + g]
                plsc.store_scatter(ob, [etv[2 + g], eiv[2 + g], bv], v)


def _body(x1t_hbm, x2t_hbm, t1_hbm, t2_hbm, pe_hbm, out_hbm,
          idx1_v, idx2_v, rows1_v, t2_v, pe_v, out_v,
          isem0, isem1, isem2, gsem0, gsem1, gsem2, osem0, osem1, osem2):
    isems = (isem0, isem1, isem2)
    gsems = (gsem0, gsem1, gsem2)
    osems = (osem0, osem1, osem2)
    wid = lax.axis_index("s") * _NC + lax.axis_index("c")
    cols = pl.ds(wid * _BT, _BT)
    pltpu.sync_copy(pe_hbm, pe_v)
    pltpu.sync_copy(t2_hbm, t2_v)
    pltpu.sync_copy(x2t_hbm.at[:, cols], idx2_v)
    iota = lax.iota(jnp.int32, _L)

    def _idx_issue(s, ib):
        pltpu.async_copy(x1t_hbm.at[s, cols], idx1_v.at[ib], isems[ib])

    def _idx_wait(s, ib):
        pltpu.make_async_copy(x1t_hbm.at[s, cols], idx1_v.at[ib],
                              isems[ib]).wait()

    def _gather(s, b):
        pltpu.async_copy(t1_hbm.at[idx1_v.at[b]], rows1_v.at[b], gsems[b])

    def _gwait(s, b):
        pltpu.make_async_copy(t1_hbm.at[idx1_v.at[b]], rows1_v.at[b],
                              gsems[b]).wait()

    def _owait(s, ob):
        pltpu.make_async_copy(out_v.at[ob, :, :, pl.ds(0, 128)],
                              out_hbm.at[s, :, wid], osems[ob]).wait()

    for s in range(_NBUF):
        _idx_issue(s, s)
    for s in range(2):
        _idx_wait(s, s)
        _gather(s, s)

    def _step(s, b, wait_pref, issue_pref):
        if wait_pref:
            _idx_wait(s + 2, (b + 2) % _NBUF)
            _gather(s + 2, (b + 2) % _NBUF)
        _gwait(s, b)
        if issue_pref:
            @pl.when(s + _NBUF < _S)
            def _():
                _idx_issue(s + _NBUF, b)

        @pl.when(s >= _NBUF)
        def _():
            _owait(s - _NBUF, b)

        _compute(rows1_v.at[b], t2_v, idx2_v, pe_v, out_v.at[b], s, iota)
        pltpu.async_copy(out_v.at[b, :, :, pl.ds(0, 128)],
                         out_hbm.at[s, :, wid], osems[b])

    main_upper = (_S - 2) // _NBUF * _NBUF

    @pl.loop(0, main_upper, step=_NBUF)
    def _main(i):
        for k in range(_NBUF):
            _step(i + k, k, True, True)

    for s in range(main_upper, _S):
        _step(s, s % _NBUF, s + 2 < _S, s + _NBUF < _S)

    for s in range(_S - _NBUF, _S):
        _owait(s, s % _NBUF)


@functools.partial(
    pl.kernel,
    out_type=jax.ShapeDtypeStruct((_S, _EMBED // 8, _B // 128, 8, 128),
                                  jnp.float32),
    mesh=plsc.VectorSubcoreMesh(core_axis_name="c", subcore_axis_name="s"),
    scratch_types=[
        pltpu.VMEM((_NBUF, _BT), jnp.int32),
        pltpu.VMEM((_S, _BT), jnp.int32),
        pltpu.VMEM((_NBUF, _BT, _HALF), jnp.float32),
        pltpu.VMEM((_V2, _HALF), jnp.float32),
        pltpu.VMEM((_S, _EMBED), jnp.float32),
        pltpu.VMEM((_NBUF, 8, 8, _OPITCH), jnp.float32),
        pltpu.SemaphoreType.DMA,
        pltpu.SemaphoreType.DMA,
        pltpu.SemaphoreType.DMA,
        pltpu.SemaphoreType.DMA,
        pltpu.SemaphoreType.DMA,
        pltpu.SemaphoreType.DMA,
        pltpu.SemaphoreType.DMA,
        pltpu.SemaphoreType.DMA,
        pltpu.SemaphoreType.DMA,
    ],
    compiler_params=pltpu.CompilerParams(use_tc_tiling_on_sc=False,
                                         needs_layout_passes=False,
                                         disable_bounds_checks=True),
)
def _emb_kernel(*refs):
    _body(*refs)


def kernel(x1, x2, table1, table2, pe):
    x1t = x1.astype(jnp.int32).T
    x2t = x2.astype(jnp.int32).T
    pe2d = pe.reshape(_S, _EMBED)
    y5 = _emb_kernel(x1t, x2t, table1, table2, pe2d)
    return jnp.transpose(y5, (2, 4, 0, 1, 3)).reshape(_B, _S, _EMBED)


# speedup vs baseline: 1.5555x; 1.5555x over previous
import functools

import jax
import jax.numpy as jnp
from jax import lax
from jax.experimental import pallas as pl
from jax.experimental.pallas import tpu as pltpu
from jax.experimental.pallas import tpu_sc as plsc

_B = 4096
_S = 200
_HALF = 32
_EMBED = 64
_NC = 2
_NS = 16
_NW = _NC * _NS
_BT = _B // _NW
_L = 16
_NBUF = 3
_OPITCH = 129


def _compute(r1, r2, pe_v, ob, s, iota):
    pev = []
    etv = []
    eiv = []
    for g in range(_EMBED // _L):
        pev.append(pe_v[s, pl.ds(g * _L, _L)])
        ev = iota + g * _L
        etv.append(ev >> 3)
        eiv.append(ev & 7)

    @plsc.parallel_loop(0, _BT, unroll=4)
    def _row(b):
        bv = jnp.full((_L,), b, jnp.int32)
        for g in range(_EMBED // _L):
            src = r1 if g < 2 else r2
            v = src[b, pl.ds((g % 2) * _L, _L)] + pev[g]
            plsc.store_scatter(ob, [etv[g], eiv[g], bv], v)


def _body(x1t_hbm, x2t_hbm, t1_hbm, t2_hbm, pe_hbm, out_hbm,
          idx1_v, idx2_v, rows1_v, rows2_v, pe_v, out_v,
          gsem0, gsem1, gsem2, osem0, osem1, osem2):
    gsems = (gsem0, gsem1, gsem2)
    osems = (osem0, osem1, osem2)
    wid = lax.axis_index("s") * _NC + lax.axis_index("c")
    cols = pl.ds(wid * _BT, _BT)
    pltpu.sync_copy(pe_hbm, pe_v)
    pltpu.sync_copy(x1t_hbm.at[:, cols], idx1_v)
    pltpu.sync_copy(x2t_hbm.at[:, cols], idx2_v)
    iota = lax.iota(jnp.int32, _L)

    def _gather(s, b):
        pltpu.async_copy(t1_hbm.at[idx1_v.at[s]], rows1_v.at[b], gsems[b])
        pltpu.async_copy(t2_hbm.at[idx2_v.at[s]], rows2_v.at[b], gsems[b])

    def _gwait(s, b):
        pltpu.make_async_copy(t1_hbm.at[idx1_v.at[s]], rows1_v.at[b],
                              gsems[b]).wait()
        pltpu.make_async_copy(t2_hbm.at[idx2_v.at[s]], rows2_v.at[b],
                              gsems[b]).wait()

    def _owait(s, b):
        pltpu.make_async_copy(out_v.at[b, :, :, pl.ds(0, 128)],
                              out_hbm.at[s, :, wid], osems[b]).wait()

    _gather(0, 0)
    _gather(1, 1)

    def _step(s, b, prefetch):
        if prefetch:
            _gather(s + 2, (b + 2) % _NBUF)
        _gwait(s, b)

        @pl.when(s >= _NBUF)
        def _():
            _owait(s - _NBUF, b)

        _compute(rows1_v.at[b], rows2_v.at[b], pe_v, out_v.at[b], s, iota)
        pltpu.async_copy(out_v.at[b, :, :, pl.ds(0, 128)],
                         out_hbm.at[s, :, wid], osems[b])

    @pl.loop(0, _S - 2, step=_NBUF)
    def _main(i):
        for b in range(_NBUF):
            _step(i + b, b, prefetch=True)

    _step(_S - 2, (_S - 2) % _NBUF, prefetch=False)
    _step(_S - 1, (_S - 1) % _NBUF, prefetch=False)

    for s in range(_S - _NBUF, _S):
        _owait(s, s % _NBUF)


@functools.partial(
    pl.kernel,
    out_type=jax.ShapeDtypeStruct((_S, _EMBED // 8, _B // 128, 8, 128),
                                  jnp.float32),
    mesh=plsc.VectorSubcoreMesh(core_axis_name="c", subcore_axis_name="s"),
    scratch_types=[
        pltpu.VMEM((_S, _BT), jnp.int32),
        pltpu.VMEM((_S, _BT), jnp.int32),
        pltpu.VMEM((_NBUF, _BT, _HALF), jnp.float32),
        pltpu.VMEM((_NBUF, _BT, _HALF), jnp.float32),
        pltpu.VMEM((_S, _EMBED), jnp.float32),
        pltpu.VMEM((_NBUF, 8, 8, _OPITCH), jnp.float32),
        pltpu.SemaphoreType.DMA,
        pltpu.SemaphoreType.DMA,
        pltpu.SemaphoreType.DMA,
        pltpu.SemaphoreType.DMA,
        pltpu.SemaphoreType.DMA,
        pltpu.SemaphoreType.DMA,
    ],
    compiler_params=pltpu.CompilerParams(use_tc_tiling_on_sc=False,
                                         needs_layout_passes=False,
                                         disable_bounds_checks=True),
)
def _emb_kernel(*refs):
    _body(*refs)


def kernel(x1, x2, table1, table2, pe):
    x1t = x1.astype(jnp.int32).T
    x2t = x2.astype(jnp.int32).T
    pe2d = pe.reshape(_S, _EMBED)
    y5 = _emb_kernel(x1t, x2t, table1, table2, pe2d)
    return jnp.transpose(y5, (2, 4, 0, 1, 3)).reshape(_B, _S, _EMBED)
